# Initial kernel scaffold; baseline (speedup 1.0000x reference)
import jax
import jax.numpy as jnp
from jax.experimental import pallas as pl


def kernel(inputs, context, W1, b1, W2, b2):
    raise NotImplementedError("write your pallas kernel here")


# speedup vs baseline: 3.0105x; 3.0105x over previous
import functools

import jax
import jax.numpy as jnp
from jax.experimental import pallas as pl

B = 262144
CTX = 16
NB = 64
H = 64
NOUT = NB + NB + (NB + 1)
NPAD = 256
LEFT, RIGHT, BOTTOM, TOP = -5.0, 5.0, -5.0, 5.0
MIN_DERIV = 0.001

BLK = 2048


def _spline_kernel(inp_ref, ctx_ref, W1_ref, b1_ref, W2_ref, b2_ref,
                   out_ref, lad_ref):
    ctx = ctx_ref[...]
    v = inp_ref[...]

    h = jnp.maximum(
        jax.lax.dot(ctx.astype(jnp.bfloat16), W1_ref[...].astype(jnp.bfloat16),
                    preferred_element_type=jnp.float32)
        + b1_ref[...], 0.0)
    params = jax.lax.dot(h.astype(jnp.bfloat16),
                         W2_ref[...].astype(jnp.bfloat16),
                         preferred_element_type=jnp.float32) + b2_ref[...]

    uw = params[:, :NB]
    uh = params[:, NB:2 * NB]
    ud = params[:, 2 * NB:]

    ew = jnp.exp(uw - jnp.max(uw, axis=1, keepdims=True))
    widths = ew * ((RIGHT - LEFT) / jnp.sum(ew, axis=1, keepdims=True))
    eh = jnp.exp(uh - jnp.max(uh, axis=1, keepdims=True))
    heights = eh * ((TOP - BOTTOM) / jnp.sum(eh, axis=1, keepdims=True))
    derivs = jax.nn.softplus(ud) + MIN_DERIV

    r = jax.lax.broadcasted_iota(jnp.int32, (NB, NB), 0)
    c = jax.lax.broadcasted_iota(jnp.int32, (NB, NB), 1)
    tri = (r <= c).astype(jnp.float32)
    cw = jax.lax.dot(widths, tri, precision=jax.lax.Precision.HIGHEST, preferred_element_type=jnp.float32)
    ch = jax.lax.dot(heights, tri, precision=jax.lax.Precision.HIGHEST, preferred_element_type=jnp.float32)

    lane = jax.lax.broadcasted_iota(jnp.int32, (1, NB), 1)
    last = lane == NB - 1
    kx = jnp.where(last, RIGHT, LEFT + cw)
    ky = jnp.where(last, TOP, BOTTOM + ch)

    cnt = (jnp.sum((kx < v).astype(jnp.int32), axis=1, keepdims=True)
           + (v > LEFT).astype(jnp.int32))
    b = jnp.clip(cnt - 1, 0, NB - 1)

    oh1 = (lane == b).astype(jnp.float32)
    oh0 = (lane == b - 1).astype(jnp.float32)
    x_k1 = jnp.sum(kx * oh1, axis=1, keepdims=True)
    y_k1 = jnp.sum(ky * oh1, axis=1, keepdims=True)
    x_k = jnp.where(b == 0, LEFT, jnp.sum(kx * oh0, axis=1, keepdims=True))
    y_k = jnp.where(b == 0, BOTTOM, jnp.sum(ky * oh0, axis=1, keepdims=True))

    lane2 = jax.lax.broadcasted_iota(jnp.int32, (1, 2 * NB), 1)
    d_k = jnp.sum(derivs * (lane2 == b).astype(jnp.float32),
                  axis=1, keepdims=True)
    d_k1 = jnp.sum(derivs * (lane2 == b + 1).astype(jnp.float32),
                   axis=1, keepdims=True)

    bin_width = x_k1 - x_k
    bin_height = y_k1 - y_k
    s_k = bin_height / bin_width
    xi = jnp.clip((v - x_k) / (bin_width + 1e-9), 0.0, 1.0)
    om = 1.0 - xi
    num_y = s_k * xi * xi + d_k * xi * om
    den_y = s_k + (d_k1 + d_k - 2.0 * s_k) * xi * om
    out_ref[...] = y_k + bin_height * (num_y / (den_y + 1e-9))

    term = d_k1 * xi * xi + 2.0 * s_k * xi * om + d_k * om * om
    deriv_num = s_k * s_k * term
    deriv_den = den_y * den_y
    lad_ref[...] = jnp.log(deriv_num + 1e-9) - jnp.log(deriv_den + 1e-9)


@jax.jit
def kernel(inputs, context, W1, b1, W2, b2):
    W2p = jnp.zeros((H, NPAD), jnp.float32).at[:, :NOUT].set(W2)
    b2p = jnp.zeros((NPAD,), jnp.float32).at[:NOUT].set(b2)

    grid = (B // BLK,)
    out, lad = pl.pallas_call(
        _spline_kernel,
        grid=grid,
        in_specs=[
            pl.BlockSpec((BLK, 1), lambda i: (i, 0)),
            pl.BlockSpec((BLK, CTX), lambda i: (i, 0)),
            pl.BlockSpec((CTX, H), lambda i: (0, 0)),
            pl.BlockSpec((1, H), lambda i: (0, 0)),
            pl.BlockSpec((H, NPAD), lambda i: (0, 0)),
            pl.BlockSpec((1, NPAD), lambda i: (0, 0)),
        ],
        out_specs=[
            pl.BlockSpec((BLK, 1), lambda i: (i, 0)),
            pl.BlockSpec((BLK, 1), lambda i: (i, 0)),
        ],
        out_shape=[
            jax.ShapeDtypeStruct((B, 1), jnp.float32),
            jax.ShapeDtypeStruct((B, 1), jnp.float32),
        ],
    )(inputs, context, W1, b1.reshape(1, H), W2p, b2p.reshape(1, NPAD))
    return out, lad[:, 0]


# speedup vs baseline: 11.9984x; 3.9855x over previous
import jax
import jax.numpy as jnp
from jax.experimental import pallas as pl

B = 262144
CTX = 16
NB = 64
H = 64
NOUT = NB + NB + (NB + 1)
NPAD = 256
ND = 72
LEFT, RIGHT, BOTTOM, TOP = -5.0, 5.0, -5.0, 5.0
MIN_DERIV = 0.001

BLK = 2048


def _spline_kernel(v_ref, ctx_ref, W1_ref, b1_ref, W2_ref, b2_ref,
                   out_ref, lad_ref):
    v = v_ref[...]
    ctxT = ctx_ref[...]

    hT = jnp.maximum(
        jax.lax.dot(W1_ref[...], ctxT, preferred_element_type=jnp.float32)
        + b1_ref[...], 0.0)
    paramsT = jax.lax.dot(W2_ref[...], hT.astype(jnp.bfloat16),
                          preferred_element_type=jnp.float32) + b2_ref[...]

    uw = paramsT[:NB]
    uh = paramsT[NB:2 * NB]
    ud = paramsT[2 * NB:2 * NB + ND]

    ew = jnp.exp(uw - jnp.max(uw, axis=0, keepdims=True))
    widths = ew * ((RIGHT - LEFT) / jnp.sum(ew, axis=0, keepdims=True))
    eh = jnp.exp(uh - jnp.max(uh, axis=0, keepdims=True))
    heights = eh * ((TOP - BOTTOM) / jnp.sum(eh, axis=0, keepdims=True))
    derivs = jax.nn.softplus(ud) + MIN_DERIV

    r = jax.lax.broadcasted_iota(jnp.int32, (NB, NB), 0)
    c = jax.lax.broadcasted_iota(jnp.int32, (NB, NB), 1)
    tri = (c <= r).astype(jnp.float32)
    cw = jax.lax.dot(tri, widths, precision=jax.lax.Precision.HIGHEST,
                     preferred_element_type=jnp.float32)
    ch = jax.lax.dot(tri, heights, precision=jax.lax.Precision.HIGHEST,
                     preferred_element_type=jnp.float32)

    sub = jax.lax.broadcasted_iota(jnp.int32, (NB, 1), 0)
    last = sub == NB - 1
    kx = jnp.where(last, RIGHT, LEFT + cw)
    ky = jnp.where(last, TOP, BOTTOM + ch)

    cnt = (jnp.sum((kx < v).astype(jnp.int32), axis=0, keepdims=True)
           + (v > LEFT).astype(jnp.int32))
    b = jnp.clip(cnt - 1, 0, NB - 1)

    m1 = (sub == b).astype(jnp.float32)
    m0 = (sub == b - 1).astype(jnp.float32)
    x_k1 = jnp.sum(kx * m1, axis=0, keepdims=True)
    y_k1 = jnp.sum(ky * m1, axis=0, keepdims=True)
    x_k = jnp.where(b == 0, LEFT, jnp.sum(kx * m0, axis=0, keepdims=True))
    y_k = jnp.where(b == 0, BOTTOM, jnp.sum(ky * m0, axis=0, keepdims=True))

    sub2 = jax.lax.broadcasted_iota(jnp.int32, (ND, 1), 0)
    d_k = jnp.sum(derivs * (sub2 == b).astype(jnp.float32),
                  axis=0, keepdims=True)
    d_k1 = jnp.sum(derivs * (sub2 == b + 1).astype(jnp.float32),
                   axis=0, keepdims=True)

    bin_width = x_k1 - x_k
    bin_height = y_k1 - y_k
    s_k = bin_height / bin_width
    xi = jnp.clip((v - x_k) / (bin_width + 1e-9), 0.0, 1.0)
    om = 1.0 - xi
    num_y = s_k * xi * xi + d_k * xi * om
    den_y = s_k + (d_k1 + d_k - 2.0 * s_k) * xi * om
    out_ref[...] = y_k + bin_height * (num_y / (den_y + 1e-9))

    term = d_k1 * xi * xi + 2.0 * s_k * xi * om + d_k * om * om
    deriv_num = s_k * s_k * term
    deriv_den = den_y * den_y
    lad_ref[...] = jnp.log(deriv_num + 1e-9) - jnp.log(deriv_den + 1e-9)


@jax.jit
def kernel(inputs, context, W1, b1, W2, b2):
    ctxT = context.T.astype(jnp.bfloat16)
    vT = inputs.reshape(1, B)
    W1T = W1.T.astype(jnp.bfloat16)
    W2p = jnp.zeros((H, NPAD), jnp.float32).at[:, :NOUT].set(W2)
    b2p = jnp.zeros((NPAD,), jnp.float32).at[:NOUT].set(b2)
    W2T = W2p.T.astype(jnp.bfloat16)

    grid = (B // BLK,)
    out, lad = pl.pallas_call(
        _spline_kernel,
        grid=grid,
        in_specs=[
            pl.BlockSpec((1, BLK), lambda i: (0, i)),
            pl.BlockSpec((CTX, BLK), lambda i: (0, i)),
            pl.BlockSpec((H, CTX), lambda i: (0, 0)),
            pl.BlockSpec((H, 1), lambda i: (0, 0)),
            pl.BlockSpec((NPAD, H), lambda i: (0, 0)),
            pl.BlockSpec((NPAD, 1), lambda i: (0, 0)),
        ],
        out_specs=[
            pl.BlockSpec((1, BLK), lambda i: (0, i)),
            pl.BlockSpec((1, BLK), lambda i: (0, i)),
        ],
        out_shape=[
            jax.ShapeDtypeStruct((1, B), jnp.float32),
            jax.ShapeDtypeStruct((1, B), jnp.float32),
        ],
    )(vT, ctxT, W1T, b1.reshape(H, 1), W2T, b2p.reshape(NPAD, 1))
    return out.reshape(B, 1), lad.reshape(B)
